# baseline (speedup 1.0000x reference)
import functools

import jax
import jax.numpy as jnp
from jax import lax
from jax.experimental import pallas as pl
from jax.experimental.pallas import tpu as pltpu
from jax.experimental.pallas import tpu_sc as plsc

_LANES = 16
_NC = 2
_NS = 16
_NW = _NC * _NS


def _pick_block(n):
    for b in (8000, 4000, 2000, 1000, 800, 400, 160, 80, 16):
        if n % b == 0:
            return b
    return None


def _body(seq_hbm, out_hbm, *refs, n, blk, nblocks, kmax):
    ins = refs[0:2]
    outs = (refs[2:6], refs[6:10])
    isems = refs[10:12]
    osems = refs[12:14]

    wid = lax.axis_index("s") * _NC + lax.axis_index("c")

    def bid(k):
        return wid + k * _NW

    def pred(k):
        return bid(k) < nblocks

    def start_in(k):
        pltpu.async_copy(seq_hbm.at[pl.ds(bid(k) * blk, blk)],
                         ins[k % 2], isems[k % 2])

    def wait_in(k):
        pltpu.make_async_copy(seq_hbm.at[pl.ds(bid(k) * blk, blk)],
                              ins[k % 2], isems[k % 2]).wait()

    def start_out(k):
        base = bid(k) * blk
        for c in range(4):
            pltpu.async_copy(outs[k % 2][c],
                             out_hbm.at[c, pl.ds(base, blk)],
                             osems[k % 2])

    def wait_out(k):
        base = bid(k) * blk
        for c in range(4):
            pltpu.make_async_copy(outs[k % 2][c],
                                  out_hbm.at[c, pl.ds(base, blk)],
                                  osems[k % 2]).wait()

    def compute(k):
        iv = ins[k % 2]
        ov = outs[k % 2]

        @plsc.parallel_loop(0, blk, step=_LANES, unroll=8)
        def inner(i):
            off = pl.multiple_of(i, _LANES)
            s = iv[pl.ds(off, _LANES)]
            one = jnp.full((_LANES,), 1.0, jnp.float32)
            nv = jnp.where(s == 4,
                           jnp.full((_LANES,), 0.25, jnp.float32),
                           jnp.zeros((_LANES,), jnp.float32))
            for c in range(4):
                ov[c][pl.ds(off, _LANES)] = jnp.where(s == c, one, nv)

    @pl.when(pred(0))
    def _():
        start_in(0)

    for k in range(kmax):
        if k + 1 < kmax:
            @pl.when(pred(k + 1))
            def _(k=k):
                start_in(k + 1)

        @pl.when(pred(k))
        def _(k=k):
            wait_in(k)
            if k >= 2:
                wait_out(k - 2)
            compute(k)
            start_out(k)

    for k in range(max(0, kmax - 2), kmax):
        @pl.when(pred(k))
        def _(k=k):
            wait_out(k)


def kernel(seq):
    n = seq.shape[0]
    blk = _pick_block(n)
    nblocks = n // blk
    kmax = -(-nblocks // _NW)
    mesh = plsc.VectorSubcoreMesh(core_axis_name="c", subcore_axis_name="s")
    f = pl.kernel(
        functools.partial(_body, n=n, blk=blk, nblocks=nblocks, kmax=kmax),
        out_type=jax.ShapeDtypeStruct((4, n), jnp.float32),
        mesh=mesh,
        compiler_params=pltpu.CompilerParams(use_tc_tiling_on_sc=False),
        scratch_types=[pltpu.VMEM((blk,), jnp.int32) for _ in range(2)]
        + [pltpu.VMEM((blk,), jnp.float32) for _ in range(8)]
        + [pltpu.SemaphoreType.DMA for _ in range(4)],
    )
    return f(seq.astype(jnp.int32))


# speedup vs baseline: 1.7056x; 1.7056x over previous
import functools

import jax
import jax.numpy as jnp
from jax import lax
from jax.experimental import pallas as pl
from jax.experimental.pallas import tpu as pltpu
from jax.experimental.pallas import tpu_sc as plsc

_LANES = 16
_TILE = 128
_NC = 2
_NS = 16
_NW = _NC * _NS
_BLK = 6400


def _body(seq_hbm, out_hbm, *refs, n, blk, nblocks, kmax):
    ins = refs[0:2]
    outs = refs[2:4]
    isems = refs[4:6]
    osems = refs[6:8]

    m = (n // _TILE) * _TILE
    nfull = m // blk
    tail = m - nfull * blk
    tail_main = tail

    wid = lax.axis_index("s") * _NC + lax.axis_index("c")

    def bid(k):
        return wid + k * _NW

    def pred(k):
        return bid(k) < nblocks

    def start_in(k, length):
        pltpu.async_copy(seq_hbm.at[pl.ds(bid(k) * blk, length)],
                         ins[k % 2].at[pl.ds(0, length)], isems[k % 2])

    def wait_in(k, length):
        pltpu.make_async_copy(seq_hbm.at[pl.ds(bid(k) * blk, length)],
                              ins[k % 2].at[pl.ds(0, length)],
                              isems[k % 2]).wait()

    def start_out_full(k):
        base = bid(k) * blk
        pltpu.async_copy(outs[k % 2],
                         out_hbm.at[:, pl.ds(base, blk)], osems[k % 2])

    def wait_out_full(k):
        base = bid(k) * blk
        pltpu.make_async_copy(outs[k % 2],
                              out_hbm.at[:, pl.ds(base, blk)],
                              osems[k % 2]).wait()

    def start_out_tail(k):
        base = nfull * blk
        pltpu.async_copy(outs[k % 2].at[:, pl.ds(0, tail_main)],
                         out_hbm.at[:, pl.ds(base, tail_main)],
                         osems[k % 2])

    def wait_out_tail(k):
        base = nfull * blk
        pltpu.make_async_copy(outs[k % 2].at[:, pl.ds(0, tail_main)],
                              out_hbm.at[:, pl.ds(base, tail_main)],
                              osems[k % 2]).wait()

    def compute(k, length):
        iv = ins[k % 2]
        ov = outs[k % 2]

        @plsc.parallel_loop(0, length, step=_LANES, unroll=8)
        def inner(i):
            off = pl.multiple_of(i, _LANES)
            s = iv[pl.ds(off, _LANES)]
            one = jnp.full((_LANES,), 1.0, jnp.float32)
            nv = jnp.where(s == 4,
                           jnp.full((_LANES,), 0.25, jnp.float32),
                           jnp.zeros((_LANES,), jnp.float32))
            for c in range(4):
                ov[c, pl.ds(off, _LANES)] = jnp.where(s == c, one, nv)

    is_tail = lambda k: bid(k) == nfull

    def start_in_any(k):
        if tail:
            @pl.when(pred(k) & jnp.logical_not(is_tail(k)))
            def _():
                start_in(k, blk)

            @pl.when(is_tail(k))
            def _():
                start_in(k, tail)
        else:
            @pl.when(pred(k))
            def _():
                start_in(k, blk)

    start_in_any(0)

    for k in range(kmax):
        if k + 1 < kmax:
            start_in_any(k + 1)

        if tail:
            @pl.when(pred(k) & jnp.logical_not(is_tail(k)))
            def _(k=k):
                wait_in(k, blk)
                if k >= 2:
                    wait_out_full(k - 2)
                compute(k, blk)
                start_out_full(k)

            @pl.when(is_tail(k))
            def _(k=k):
                wait_in(k, tail)
                if k >= 2:
                    wait_out_full(k - 2)
                compute(k, tail)
                start_out_tail(k)
        else:
            @pl.when(pred(k))
            def _(k=k):
                wait_in(k, blk)
                if k >= 2:
                    wait_out_full(k - 2)
                compute(k, blk)
                start_out_full(k)

    for k in range(max(0, kmax - 2), kmax):
        if tail:
            @pl.when(pred(k) & jnp.logical_not(is_tail(k)))
            def _(k=k):
                wait_out_full(k)

            @pl.when(is_tail(k))
            def _(k=k):
                wait_out_tail(k)
        else:
            @pl.when(pred(k))
            def _(k=k):
                wait_out_full(k)


def kernel(seq):
    n = seq.shape[0]
    blk = _BLK
    m = (n // _TILE) * _TILE
    nblocks = -(-m // blk)
    kmax = -(-nblocks // _NW)
    mesh = plsc.VectorSubcoreMesh(core_axis_name="c", subcore_axis_name="s")
    f = pl.kernel(
        functools.partial(_body, n=n, blk=blk, nblocks=nblocks, kmax=kmax),
        out_type=jax.ShapeDtypeStruct((4, n), jnp.float32),
        mesh=mesh,
        scratch_types=[pltpu.VMEM((blk,), jnp.int32) for _ in range(2)]
        + [pltpu.VMEM((4, blk), jnp.float32) for _ in range(2)]
        + [pltpu.SemaphoreType.DMA for _ in range(4)],
    )
    seq = seq.astype(jnp.int32)
    out = f(seq)
    if m < n:
        rem = seq[m:]
        cls = jnp.arange(4, dtype=jnp.int32)[:, None]
        patch = jnp.where(rem[None, :] == cls, jnp.float32(1.0),
                          jnp.where(rem[None, :] == 4,
                                    jnp.float32(0.25), jnp.float32(0.0)))
        out = lax.dynamic_update_slice(out, patch, (0, m))
    return out


# speedup vs baseline: 1.7662x; 1.0355x over previous
import functools

import jax
import jax.numpy as jnp
from jax import lax
from jax.experimental import pallas as pl
from jax.experimental.pallas import tpu as pltpu
from jax.experimental.pallas import tpu_sc as plsc

_LANES = 16
_TILE = 128
_NC = 2
_NS = 16
_NW = _NC * _NS
_BLK = 6400


def _body(seq_hbm, out_hbm, in0, in1, ov0, ov1, is0, is1, os0, os1,
          *, n, blk, nblocks, kmax):
    m = (n // _TILE) * _TILE
    nfull = m // blk
    tail = m - nfull * blk

    wid = lax.axis_index("s") * _NC + lax.axis_index("c")

    def bid(k):
        return wid + k * _NW

    def pred(k):
        return bid(k) < nblocks

    def is_tail(k):
        return bid(k) == nfull

    def start_in(k, iv, isem):
        if tail:
            @pl.when(pred(k) & jnp.logical_not(is_tail(k)))
            def _():
                pltpu.async_copy(seq_hbm.at[pl.ds(bid(k) * blk, blk)],
                                 iv, isem)

            @pl.when(is_tail(k))
            def _():
                pltpu.async_copy(seq_hbm.at[pl.ds(nfull * blk, tail)],
                                 iv.at[pl.ds(0, tail)], isem)
        else:
            @pl.when(pred(k))
            def _():
                pltpu.async_copy(seq_hbm.at[pl.ds(bid(k) * blk, blk)],
                                 iv, isem)

    def wait_in(k, iv, isem):
        if tail:
            @pl.when(jnp.logical_not(is_tail(k)))
            def _():
                pltpu.make_async_copy(seq_hbm.at[pl.ds(bid(k) * blk, blk)],
                                      iv, isem).wait()

            @pl.when(is_tail(k))
            def _():
                pltpu.make_async_copy(seq_hbm.at[pl.ds(nfull * blk, tail)],
                                      iv.at[pl.ds(0, tail)], isem).wait()
        else:
            pltpu.make_async_copy(seq_hbm.at[pl.ds(bid(k) * blk, blk)],
                                  iv, isem).wait()

    def start_out(k, ov, osem):
        if tail:
            @pl.when(jnp.logical_not(is_tail(k)))
            def _():
                pltpu.async_copy(ov, out_hbm.at[:, pl.ds(bid(k) * blk, blk)],
                                 osem)

            @pl.when(is_tail(k))
            def _():
                pltpu.async_copy(ov.at[:, pl.ds(0, tail)],
                                 out_hbm.at[:, pl.ds(nfull * blk, tail)],
                                 osem)
        else:
            pltpu.async_copy(ov, out_hbm.at[:, pl.ds(bid(k) * blk, blk)],
                             osem)

    def wait_out(k, ov, osem):
        if tail:
            @pl.when(jnp.logical_not(is_tail(k)))
            def _():
                pltpu.make_async_copy(
                    ov, out_hbm.at[:, pl.ds(bid(k) * blk, blk)], osem).wait()

            @pl.when(is_tail(k))
            def _():
                pltpu.make_async_copy(
                    ov.at[:, pl.ds(0, tail)],
                    out_hbm.at[:, pl.ds(nfull * blk, tail)], osem).wait()
        else:
            pltpu.make_async_copy(
                ov, out_hbm.at[:, pl.ds(bid(k) * blk, blk)], osem).wait()

    def compute(iv, ov):
        @plsc.parallel_loop(0, blk, step=_LANES, unroll=8)
        def inner(i):
            off = pl.multiple_of(i, _LANES)
            s = iv[pl.ds(off, _LANES)]
            one = jnp.full((_LANES,), 1.0, jnp.float32)
            nv = jnp.where(s == 4,
                           jnp.full((_LANES,), 0.25, jnp.float32),
                           jnp.zeros((_LANES,), jnp.float32))
            for c in range(4):
                ov[c, pl.ds(off, _LANES)] = jnp.where(s == c, one, nv)

    def iteration(k, cur, nxt):
        (iv, ov, isem, osem) = cur
        start_in(k + 1, nxt[0], nxt[2])

        @pl.when(pred(k))
        def _():
            wait_in(k, iv, isem)

            @pl.when(k >= 2)
            def _():
                wait_out(k - 2, ov, osem)

            compute(iv, ov)
            start_out(k, ov, osem)

    bufs = ((in0, ov0, is0, os0), (in1, ov1, is1, os1))

    start_in(0, in0, is0)

    def step(k, carry):
        @pl.when(lax.rem(k, 2) == 0)
        def _():
            iteration(k, bufs[0], bufs[1])

        @pl.when(lax.rem(k, 2) == 1)
        def _():
            iteration(k, bufs[1], bufs[0])

        return carry

    lax.fori_loop(0, kmax, step, 0)

    for j in range(max(0, kmax - 3), kmax):
        @pl.when(pred(j) & jnp.logical_not(pred(j + 2)))
        def _(j=j):
            wait_out(j, bufs[j % 2][1], bufs[j % 2][3])


def kernel(seq):
    n = seq.shape[0]
    blk = _BLK
    m = (n // _TILE) * _TILE
    nblocks = -(-m // blk)
    kmax = -(-nblocks // _NW)
    mesh = plsc.VectorSubcoreMesh(core_axis_name="c", subcore_axis_name="s")
    f = pl.kernel(
        functools.partial(_body, n=n, blk=blk, nblocks=nblocks, kmax=kmax),
        out_type=jax.ShapeDtypeStruct((4, n), jnp.float32),
        mesh=mesh,
        scratch_types=[pltpu.VMEM((blk,), jnp.int32) for _ in range(2)]
        + [pltpu.VMEM((4, blk), jnp.float32) for _ in range(2)]
        + [pltpu.SemaphoreType.DMA for _ in range(4)],
    )
    seq = seq.astype(jnp.int32)
    out = f(seq)
    if m < n:
        rem = seq[m:]
        cls = jnp.arange(4, dtype=jnp.int32)[:, None]
        patch = jnp.where(rem[None, :] == cls, jnp.float32(1.0),
                          jnp.where(rem[None, :] == 4,
                                    jnp.float32(0.25), jnp.float32(0.0)))
        out = lax.dynamic_update_slice(out, patch, (0, m))
    return out


# speedup vs baseline: 1.9206x; 1.0874x over previous
import functools

import jax
import jax.numpy as jnp
from jax import lax
from jax.experimental import pallas as pl
from jax.experimental.pallas import tpu as pltpu
from jax.experimental.pallas import tpu_sc as plsc

_LANES = 16
_TILE = 128
_NC = 2
_NS = 16
_NW = _NC * _NS
_BLK = 6400


def _body(seq_hbm, out_hbm, in0, in1, ov0, ov1, is0, is1, os0, os1,
          *, n, blk, nblocks, kmax):
    m = (n // _TILE) * _TILE
    nfull = m // blk
    tail = m - nfull * blk

    wid = lax.axis_index("s") * _NC + lax.axis_index("c")

    def bid(k):
        return wid + k * _NW

    def pred(k):
        return bid(k) < nblocks

    def is_tail(k):
        return bid(k) == nfull

    def start_in(k, iv, isem):
        if tail:
            @pl.when(pred(k) & jnp.logical_not(is_tail(k)))
            def _():
                pltpu.async_copy(seq_hbm.at[pl.ds(bid(k) * blk, blk)],
                                 iv, isem)

            @pl.when(is_tail(k))
            def _():
                pltpu.async_copy(seq_hbm.at[pl.ds(nfull * blk, tail)],
                                 iv.at[pl.ds(0, tail)], isem)
        else:
            @pl.when(pred(k))
            def _():
                pltpu.async_copy(seq_hbm.at[pl.ds(bid(k) * blk, blk)],
                                 iv, isem)

    def wait_in(k, iv, isem):
        if tail:
            @pl.when(jnp.logical_not(is_tail(k)))
            def _():
                pltpu.make_async_copy(seq_hbm.at[pl.ds(bid(k) * blk, blk)],
                                      iv, isem).wait()

            @pl.when(is_tail(k))
            def _():
                pltpu.make_async_copy(seq_hbm.at[pl.ds(nfull * blk, tail)],
                                      iv.at[pl.ds(0, tail)], isem).wait()
        else:
            pltpu.make_async_copy(seq_hbm.at[pl.ds(bid(k) * blk, blk)],
                                  iv, isem).wait()

    def start_out(k, ov, osem):
        if tail:
            @pl.when(jnp.logical_not(is_tail(k)))
            def _():
                pltpu.async_copy(ov, out_hbm.at[:, pl.ds(bid(k) * blk, blk)],
                                 osem)

            @pl.when(is_tail(k))
            def _():
                pltpu.async_copy(ov.at[:, pl.ds(0, tail)],
                                 out_hbm.at[:, pl.ds(nfull * blk, tail)],
                                 osem)
        else:
            pltpu.async_copy(ov, out_hbm.at[:, pl.ds(bid(k) * blk, blk)],
                             osem)

    def wait_out(k, ov, osem):
        if tail:
            @pl.when(jnp.logical_not(is_tail(k)))
            def _():
                pltpu.make_async_copy(
                    ov, out_hbm.at[:, pl.ds(bid(k) * blk, blk)], osem).wait()

            @pl.when(is_tail(k))
            def _():
                pltpu.make_async_copy(
                    ov.at[:, pl.ds(0, tail)],
                    out_hbm.at[:, pl.ds(nfull * blk, tail)], osem).wait()
        else:
            pltpu.make_async_copy(
                ov, out_hbm.at[:, pl.ds(bid(k) * blk, blk)], osem).wait()

    def compute(iv, ov):
        @plsc.parallel_loop(0, blk, step=_LANES, unroll=4)
        def inner(i):
            off = pl.multiple_of(i, _LANES)
            s = iv[pl.ds(off, _LANES)]
            one = jnp.full((_LANES,), 1.0, jnp.float32)
            nv = jnp.where(s == 4,
                           jnp.full((_LANES,), 0.25, jnp.float32),
                           jnp.zeros((_LANES,), jnp.float32))
            for c in range(4):
                ov[c, pl.ds(off, _LANES)] = jnp.where(s == c, one, nv)

    def iteration(k, cur, nxt):
        (iv, ov, isem, osem) = cur
        start_in(k + 1, nxt[0], nxt[2])

        @pl.when(pred(k))
        def _():
            wait_in(k, iv, isem)

            @pl.when(k >= 2)
            def _():
                wait_out(k - 2, ov, osem)

            compute(iv, ov)
            start_out(k, ov, osem)

    bufs = ((in0, ov0, is0, os0), (in1, ov1, is1, os1))

    start_in(0, in0, is0)

    def step(k, carry):
        @pl.when(lax.rem(k, 2) == 0)
        def _():
            iteration(k, bufs[0], bufs[1])

        @pl.when(lax.rem(k, 2) == 1)
        def _():
            iteration(k, bufs[1], bufs[0])

        return carry

    lax.fori_loop(0, kmax, step, 0)

    for j in range(max(0, kmax - 3), kmax):
        @pl.when(pred(j) & jnp.logical_not(pred(j + 2)))
        def _(j=j):
            wait_out(j, bufs[j % 2][1], bufs[j % 2][3])


def kernel(seq):
    n = seq.shape[0]
    blk = _BLK
    m = (n // _TILE) * _TILE
    nblocks = -(-m // blk)
    kmax = -(-nblocks // _NW)
    mesh = plsc.VectorSubcoreMesh(core_axis_name="c", subcore_axis_name="s")
    f = pl.kernel(
        functools.partial(_body, n=n, blk=blk, nblocks=nblocks, kmax=kmax),
        out_type=jax.ShapeDtypeStruct((4, n), jnp.float32),
        mesh=mesh,
        scratch_types=[pltpu.VMEM((blk,), jnp.int32) for _ in range(2)]
        + [pltpu.VMEM((4, blk), jnp.float32) for _ in range(2)]
        + [pltpu.SemaphoreType.DMA for _ in range(4)],
    )
    seq = seq.astype(jnp.int32)
    out = f(seq)
    if m < n:
        rem = seq[m:]
        cls = jnp.arange(4, dtype=jnp.int32)[:, None]
        patch = jnp.where(rem[None, :] == cls, jnp.float32(1.0),
                          jnp.where(rem[None, :] == 4,
                                    jnp.float32(0.25), jnp.float32(0.0)))
        out = lax.dynamic_update_slice(out, patch, (0, m))
    return out
